# baseline (speedup 1.0000x reference)
import functools

import jax
import jax.numpy as jnp
from jax import lax
from jax.experimental import pallas as pl
from jax.experimental.pallas import tpu as pltpu
from jax.experimental.pallas import tpu_sc as plsc

B = 16384
C = 1000

LAMB = max(5.0, 1500.0 / (1.0 + 0.1 * 1))
INV = 1.0 / (1.0 + LAMB)

_NC, _NS, _L = 2, 16, 16
_NW = _NC * _NS
_BPW = B // _NW
_CHUNK = 128
_NCH = _BPW // _CHUNK

@functools.cache
def _make_sc_gather():
    mesh = plsc.VectorSubcoreMesh(
        core_axis_name="c", subcore_axis_name="s",
        num_cores=_NC, num_subcores=_NS,
    )

    @functools.partial(
        pl.kernel,
        out_type=[
            jax.ShapeDtypeStruct((_NW, _NCH, _CHUNK), jnp.float32),
            jax.ShapeDtypeStruct((_NW, _NCH, _CHUNK), jnp.float32),
        ],
        mesh=mesh,
        scratch_types=[
            pltpu.VMEM((_BPW,), jnp.int32),
            pltpu.VMEM((_NCH, _CHUNK), jnp.int32),
            pltpu.VMEM((_NCH, _CHUNK), jnp.float32),
            pltpu.VMEM((_NCH, _CHUNK), jnp.float32),
            pltpu.SemaphoreType.DMA,
        ],
    )
    def _sc_gather(cos_hbm, phi_hbm, tgt_hbm, ct_out, pt_out,
                   tgt_v, idx_v, cg_v, pg_v, sem):
        wid = lax.axis_index("s") * _NC + lax.axis_index("c")
        base = wid * _BPW
        pltpu.sync_copy(tgt_hbm.at[pl.ds(base, _BPW)], tgt_v)
        lane = lax.broadcasted_iota(jnp.int32, (_L,), 0)
        for ch in range(_NCH):
            for j in range(_CHUNK // _L):
                off = ch * _CHUNK + j * _L
                t16 = tgt_v[pl.ds(off, _L)]
                rows = (base + off) + lane
                idx_v[ch, pl.ds(j * _L, _L)] = rows * C + t16
        for ch in range(_NCH):
            pltpu.async_copy(cos_hbm.at[idx_v.at[ch]], cg_v.at[ch], sem).wait()
            pltpu.async_copy(phi_hbm.at[idx_v.at[ch]], pg_v.at[ch], sem).wait()
        pltpu.sync_copy(cg_v, ct_out.at[wid])
        pltpu.sync_copy(pg_v, pt_out.at[wid])

    return _sc_gather


_R = 256
_NT = B // _R


def _rows_body(cos_ref, m_ref, s_ref):
    x = cos_ref[...]
    m = jnp.max(x, axis=1, keepdims=True)
    s = jnp.sum(jnp.exp(x - m), axis=1, keepdims=True)
    m_ref[...] = m
    s_ref[...] = s


_rows_call = pl.pallas_call(
    _rows_body,
    grid=(_NT,),
    in_specs=[pl.BlockSpec((_R, C), lambda i: (i, 0))],
    out_specs=[
        pl.BlockSpec((_R, 1), lambda i: (i, 0)),
        pl.BlockSpec((_R, 1), lambda i: (i, 0)),
    ],
    out_shape=[
        jax.ShapeDtypeStruct((B, 1), jnp.float32),
        jax.ShapeDtypeStruct((B, 1), jnp.float32),
    ],
)


def _combine_body(m_ref, s_ref, ct_ref, pt_ref, out_ref):
    M = m_ref[...]
    S0 = s_ref[...]
    ct = ct_ref[...]
    pt = pt_ref[...]
    out_t = ct - ct * INV + pt * INV
    m = jnp.maximum(M, out_t)
    S = S0 * jnp.exp(M - m) - jnp.exp(ct - m) + jnp.exp(out_t - m)
    loss = m + jnp.log(S) - out_t
    out_ref[...] = jnp.sum(loss, keepdims=True) * (1.0 / B)


_combine_call = pl.pallas_call(
    _combine_body,
    out_shape=jax.ShapeDtypeStruct((1, 1), jnp.float32),
)


def kernel(cos_theta, phi_theta, target):
    tgt = target.reshape(-1).astype(jnp.int32)
    cos_flat = cos_theta.reshape(-1)
    phi_flat = phi_theta.reshape(-1)
    ct, pt = _make_sc_gather()(cos_flat, phi_flat, tgt)
    m, s0 = _rows_call(cos_theta)
    out = _combine_call(
        m.reshape(128, 128),
        s0.reshape(128, 128),
        ct.reshape(128, 128),
        pt.reshape(128, 128),
    )
    return out.reshape(())


# speedup vs baseline: 1.1575x; 1.1575x over previous
import functools

import jax
import jax.numpy as jnp
from jax import lax
from jax.experimental import pallas as pl
from jax.experimental.pallas import tpu as pltpu
from jax.experimental.pallas import tpu_sc as plsc

B = 16384
C = 1000

LAMB = max(5.0, 1500.0 / (1.0 + 0.1 * 1))
INV = 1.0 / (1.0 + LAMB)

_NC, _NS, _L = 2, 16, 16
_NW = _NC * _NS
_BPW = B // _NW
_CHUNK = 128
_NCH = _BPW // _CHUNK


@functools.cache
def _make_sc_gather():
    mesh = plsc.VectorSubcoreMesh(
        core_axis_name="c", subcore_axis_name="s",
        num_cores=_NC, num_subcores=_NS,
    )

    @functools.partial(
        pl.kernel,
        out_type=jax.ShapeDtypeStruct((_NW, _NCH, _CHUNK), jnp.float32),
        mesh=mesh,
        scratch_types=[
            pltpu.VMEM((_BPW,), jnp.int32),
            pltpu.VMEM((_NCH, _CHUNK), jnp.int32),
            pltpu.VMEM((_NCH, _CHUNK), jnp.float32),
            pltpu.SemaphoreType.DMA,
        ],
    )
    def _sc_gather(phi_hbm, tgt_hbm, pt_out, tgt_v, idx_v, pg_v, sem):
        wid = lax.axis_index("s") * _NC + lax.axis_index("c")
        base = wid * _BPW
        pltpu.sync_copy(tgt_hbm.at[pl.ds(base, _BPW)], tgt_v)
        lane = lax.broadcasted_iota(jnp.int32, (_L,), 0)
        for ch in range(_NCH):
            for j in range(_CHUNK // _L):
                off = ch * _CHUNK + j * _L
                t16 = tgt_v[pl.ds(off, _L)]
                rows = (base + off) + lane
                idx_v[ch, pl.ds(j * _L, _L)] = rows * C + t16
        copies = [
            pltpu.async_copy(phi_hbm.at[idx_v.at[ch]], pg_v.at[ch], sem)
            for ch in range(_NCH)
        ]
        for cp in copies:
            cp.wait()
        pltpu.sync_copy(pg_v, pt_out.at[wid])

    return _sc_gather


_R = 256
_NT = B // _R


def _rows_body(cos_ref, tgt_ref, m_ref, s_ref, ct_ref):
    x = cos_ref[...]
    t = tgt_ref[...]
    m = jnp.max(x, axis=1, keepdims=True)
    s = jnp.sum(jnp.exp(x - m), axis=1, keepdims=True)
    col = lax.broadcasted_iota(jnp.int32, x.shape, 1)
    ct = jnp.sum(jnp.where(col == t, x, 0.0), axis=1, keepdims=True)
    m_ref[...] = m
    s_ref[...] = s
    ct_ref[...] = ct


_rows_call = pl.pallas_call(
    _rows_body,
    grid=(_NT,),
    in_specs=[
        pl.BlockSpec((_R, C), lambda i: (i, 0)),
        pl.BlockSpec((_R, 1), lambda i: (i, 0)),
    ],
    out_specs=[
        pl.BlockSpec((_R, 1), lambda i: (i, 0)),
        pl.BlockSpec((_R, 1), lambda i: (i, 0)),
        pl.BlockSpec((_R, 1), lambda i: (i, 0)),
    ],
    out_shape=[
        jax.ShapeDtypeStruct((B, 1), jnp.float32),
        jax.ShapeDtypeStruct((B, 1), jnp.float32),
        jax.ShapeDtypeStruct((B, 1), jnp.float32),
    ],
)


def _combine_body(m_ref, s_ref, ct_ref, pt_ref, out_ref):
    M = m_ref[...]
    S0 = s_ref[...]
    ct = ct_ref[...]
    pt = pt_ref[...]
    out_t = ct - ct * INV + pt * INV
    m = jnp.maximum(M, out_t)
    S = S0 * jnp.exp(M - m) - jnp.exp(ct - m) + jnp.exp(out_t - m)
    loss = m + jnp.log(S) - out_t
    out_ref[...] = jnp.sum(loss, keepdims=True) * (1.0 / B)


_combine_call = pl.pallas_call(
    _combine_body,
    out_shape=jax.ShapeDtypeStruct((1, 1), jnp.float32),
)


def kernel(cos_theta, phi_theta, target):
    tgt = target.reshape(-1).astype(jnp.int32)
    pt = _make_sc_gather()(phi_theta.reshape(-1), tgt)
    m, s0, ct = _rows_call(cos_theta, tgt.reshape(B, 1))
    out = _combine_call(
        m.reshape(128, 128),
        s0.reshape(128, 128),
        ct.reshape(128, 128),
        pt.reshape(128, 128),
    )
    return out.reshape(())


# speedup vs baseline: 1.1600x; 1.0022x over previous
import functools

import jax
import jax.numpy as jnp
from jax import lax
from jax.experimental import pallas as pl
from jax.experimental.pallas import tpu as pltpu
from jax.experimental.pallas import tpu_sc as plsc

B = 16384
C = 1000

LAMB = max(5.0, 1500.0 / (1.0 + 0.1 * 1))
INV = 1.0 / (1.0 + LAMB)

_NC, _NS, _L = 2, 16, 16
_NW = _NC * _NS
_BPW = B // _NW


@functools.cache
def _make_sc_gather():
    mesh = plsc.VectorSubcoreMesh(
        core_axis_name="c", subcore_axis_name="s",
        num_cores=_NC, num_subcores=_NS,
    )

    @functools.partial(
        pl.kernel,
        out_type=jax.ShapeDtypeStruct((B,), jnp.float32),
        mesh=mesh,
        scratch_types=[
            pltpu.VMEM((_BPW,), jnp.int32),
            pltpu.VMEM((_BPW,), jnp.int32),
            pltpu.VMEM((_BPW,), jnp.float32),
            pltpu.SemaphoreType.DMA,
        ],
    )
    def _sc_gather(phi_hbm, tgt_hbm, pt_out, tgt_v, fidx_v, pg_v, sem):
        wid = lax.axis_index("s") * _NC + lax.axis_index("c")
        base = wid * _BPW
        pltpu.sync_copy(tgt_hbm.at[pl.ds(base, _BPW)], tgt_v)
        lane = lax.broadcasted_iota(jnp.int32, (_L,), 0)
        for g in range(_BPW // _L):
            t16 = tgt_v[pl.ds(g * _L, _L)]
            fidx_v[pl.ds(g * _L, _L)] = (base + g * _L + lane) * jnp.int32(C) + t16
        pltpu.async_copy(phi_hbm.at[fidx_v], pg_v, sem).wait()
        pltpu.sync_copy(pg_v, pt_out.at[pl.ds(base, _BPW)])

    return _sc_gather


_R = 256
_NT = B // _R


def _rows_body(cos_ref, tgt_ref, m_ref, s_ref, ct_ref):
    x = cos_ref[...]
    t = tgt_ref[...]
    m = jnp.max(x, axis=1, keepdims=True)
    s = jnp.sum(jnp.exp(x - m), axis=1, keepdims=True)
    col = lax.broadcasted_iota(jnp.int32, x.shape, 1)
    ct = jnp.sum(jnp.where(col == t, x, 0.0), axis=1, keepdims=True)
    m_ref[...] = m
    s_ref[...] = s
    ct_ref[...] = ct


_rows_call = pl.pallas_call(
    _rows_body,
    grid=(_NT,),
    in_specs=[
        pl.BlockSpec((_R, C), lambda i: (i, 0)),
        pl.BlockSpec((_R, 1), lambda i: (i, 0)),
    ],
    out_specs=[
        pl.BlockSpec((_R, 1), lambda i: (i, 0)),
        pl.BlockSpec((_R, 1), lambda i: (i, 0)),
        pl.BlockSpec((_R, 1), lambda i: (i, 0)),
    ],
    out_shape=[
        jax.ShapeDtypeStruct((B, 1), jnp.float32),
        jax.ShapeDtypeStruct((B, 1), jnp.float32),
        jax.ShapeDtypeStruct((B, 1), jnp.float32),
    ],
)


def _combine_body(m_ref, s_ref, ct_ref, pt_ref, out_ref):
    M = m_ref[...]
    S0 = s_ref[...]
    ct = ct_ref[...]
    pt = pt_ref[...]
    out_t = ct - ct * INV + pt * INV
    m = jnp.maximum(M, out_t)
    S = S0 * jnp.exp(M - m) - jnp.exp(ct - m) + jnp.exp(out_t - m)
    loss = m + jnp.log(S) - out_t
    out_ref[...] = jnp.sum(loss, keepdims=True) * (1.0 / B)


_combine_call = pl.pallas_call(
    _combine_body,
    out_shape=jax.ShapeDtypeStruct((1, 1), jnp.float32),
)


def kernel(cos_theta, phi_theta, target):
    tgt = target.reshape(-1).astype(jnp.int32)
    pt = _make_sc_gather()(phi_theta.reshape(B * C), tgt)
    m, s0, ct = _rows_call(cos_theta, tgt.reshape(B, 1))
    out = _combine_call(
        m.reshape(128, 128),
        s0.reshape(128, 128),
        ct.reshape(128, 128),
        pt.reshape(128, 128),
    )
    return out.reshape(())


# speedup vs baseline: 1.8739x; 1.6154x over previous
import jax
import jax.numpy as jnp
from jax import lax
from jax.experimental import pallas as pl

B = 16384
C = 1000

LAMB = max(5.0, 1500.0 / (1.0 + 0.1 * 1))
INV = 1.0 / (1.0 + LAMB)

_R = 512
_NT = B // _R


def _body(cos_ref, phi_ref, tgt_ref, out_ref):
    x = cos_ref[...]
    p = phi_ref[...]
    t = tgt_ref[...]
    col = lax.broadcasted_iota(jnp.int32, x.shape, 1)
    onehot = col == t
    out = jnp.where(onehot, x - x * INV + p * INV, x)
    m = jnp.max(out, axis=1, keepdims=True)
    s = jnp.sum(jnp.exp(out - m), axis=1, keepdims=True)
    out_t = jnp.sum(jnp.where(onehot, out, 0.0), axis=1, keepdims=True)
    tile_loss = jnp.sum(m + jnp.log(s) - out_t, axis=0, keepdims=True) * (1.0 / B)

    @pl.when(pl.program_id(0) == 0)
    def _():
        out_ref[...] = jnp.zeros_like(out_ref)

    out_ref[...] += tile_loss


_call = pl.pallas_call(
    _body,
    grid=(_NT,),
    in_specs=[
        pl.BlockSpec((_R, C), lambda i: (i, 0)),
        pl.BlockSpec((_R, C), lambda i: (i, 0)),
        pl.BlockSpec((_R, 1), lambda i: (i, 0)),
    ],
    out_specs=pl.BlockSpec((1, 1), lambda i: (0, 0)),
    out_shape=jax.ShapeDtypeStruct((1, 1), jnp.float32),
)


def kernel(cos_theta, phi_theta, target):
    tgt = target.reshape(-1).astype(jnp.int32)
    out = _call(cos_theta, phi_theta, tgt.reshape(B, 1))
    return out.reshape(())


# speedup vs baseline: 1.9619x; 1.0470x over previous
import jax
import jax.numpy as jnp
from jax import lax
from jax.experimental import pallas as pl

B = 16384
C = 1000

LAMB = max(5.0, 1500.0 / (1.0 + 0.1 * 1))
INV = 1.0 / (1.0 + LAMB)

_R = 1024
_NT = B // _R


def _body(cos_ref, phi_ref, tgt_ref, out_ref):
    x = cos_ref[...]
    p = phi_ref[...]
    t = tgt_ref[...]
    col = lax.broadcasted_iota(jnp.int32, x.shape, 1)
    onehot = col == t
    out = jnp.where(onehot, x - x * INV + p * INV, x)
    m = jnp.max(out, axis=1, keepdims=True)
    s = jnp.sum(jnp.exp(out - m), axis=1, keepdims=True)
    out_t = jnp.sum(jnp.where(onehot, out, 0.0), axis=1, keepdims=True)
    tile_loss = jnp.sum(m + jnp.log(s) - out_t, axis=0, keepdims=True) * (1.0 / B)

    @pl.when(pl.program_id(0) == 0)
    def _():
        out_ref[...] = jnp.zeros_like(out_ref)

    out_ref[...] += tile_loss


_call = pl.pallas_call(
    _body,
    grid=(_NT,),
    in_specs=[
        pl.BlockSpec((_R, C), lambda i: (i, 0)),
        pl.BlockSpec((_R, C), lambda i: (i, 0)),
        pl.BlockSpec((_R, 1), lambda i: (i, 0)),
    ],
    out_specs=pl.BlockSpec((1, 1), lambda i: (0, 0)),
    out_shape=jax.ShapeDtypeStruct((1, 1), jnp.float32),
)


def kernel(cos_theta, phi_theta, target):
    tgt = target.reshape(-1).astype(jnp.int32)
    out = _call(cos_theta, phi_theta, tgt.reshape(B, 1))
    return out.reshape(())


# speedup vs baseline: 1.9834x; 1.0110x over previous
import jax
import jax.numpy as jnp
from jax import lax
from jax.experimental import pallas as pl

B = 16384
C = 1000

LAMB = max(5.0, 1500.0 / (1.0 + 0.1 * 1))
INV = 1.0 / (1.0 + LAMB)

_R = 2048
_NT = B // _R


def _body(cos_ref, phi_ref, tgt_ref, out_ref):
    x = cos_ref[...]
    p = phi_ref[...]
    t = tgt_ref[...]
    col = lax.broadcasted_iota(jnp.int32, x.shape, 1)
    onehot = col == t
    out = jnp.where(onehot, x - x * INV + p * INV, x)
    m = jnp.max(out, axis=1, keepdims=True)
    s = jnp.sum(jnp.exp(out - m), axis=1, keepdims=True)
    out_t = jnp.sum(jnp.where(onehot, out, 0.0), axis=1, keepdims=True)
    tile_loss = jnp.sum(m + jnp.log(s) - out_t, axis=0, keepdims=True) * (1.0 / B)

    @pl.when(pl.program_id(0) == 0)
    def _():
        out_ref[...] = jnp.zeros_like(out_ref)

    out_ref[...] += tile_loss


_call = pl.pallas_call(
    _body,
    grid=(_NT,),
    in_specs=[
        pl.BlockSpec((_R, C), lambda i: (i, 0)),
        pl.BlockSpec((_R, C), lambda i: (i, 0)),
        pl.BlockSpec((_R, 1), lambda i: (i, 0)),
    ],
    out_specs=pl.BlockSpec((1, 1), lambda i: (0, 0)),
    out_shape=jax.ShapeDtypeStruct((1, 1), jnp.float32),
)


def kernel(cos_theta, phi_theta, target):
    tgt = target.reshape(-1).astype(jnp.int32)
    out = _call(cos_theta, phi_theta, tgt.reshape(B, 1))
    return out.reshape(())
